# Initial kernel scaffold; baseline (speedup 1.0000x reference)
import jax
import jax.numpy as jnp
from jax.experimental import pallas as pl


def kernel(x, edge_index, ln_g, ln_b, W_in, b_in, W1, b1, W2, b2):
    raise NotImplementedError("write your pallas kernel here")


# speedup vs baseline: 3.4673x; 3.4673x over previous
import functools

import jax
import jax.numpy as jnp
from jax import lax
from jax.experimental import pallas as pl
from jax.experimental.pallas import tpu as pltpu
from jax.experimental.pallas import tpu_sc as plsc

N = 10000
D = 128
BETA = 0.5
NC = 2
NS = 16
NW = NC * NS
ROW = 128
ICH = 8
NP = 10240
ZCH = NP // NS
BLK = 1000
GRID = N // BLK


def _sc_agg_body(with_deg, m_hbm, src_hbm, dst_hbm, zacc_hbm, *rest):
    if with_deg:
        (zdeg_hbm, agg_out, deg_out, acc_sh, deg_sh,
         srcb, dstb, rows0, rows1, ones_v, gsem0, gsem1) = rest
    else:
        zdeg_hbm = deg_out = deg_sh = ones_v = None
        (agg_out, acc_sh,
         srcb, dstb, rows0, rows1, gsem0, gsem1) = rest

    c = lax.axis_index("c")
    s = lax.axis_index("s")
    w = c * NS + s
    nrows = src_hbm.shape[0] // NW
    ch = srcb.shape[0]
    base = w * nrows

    pltpu.sync_copy(zacc_hbm.at[pl.ds(s * ZCH, ZCH)],
                    acc_sh.at[pl.ds(s * ZCH, ZCH)])
    if with_deg:
        pltpu.sync_copy(zdeg_hbm.at[pl.ds(s * ZCH, ZCH)],
                        deg_sh.at[pl.ds(s * ZCH, ZCH)])
        for k in range(ROW // 16):
            ones_v[pl.ds(k * 16, 16)] = jnp.ones((16,), jnp.float32)

    plsc.subcore_barrier()

    def chunk_body(ci, carry):
        rb = base + ci * ch
        pltpu.sync_copy(src_hbm.at[pl.ds(rb, ch)], srcb)
        pltpu.sync_copy(dst_hbm.at[pl.ds(rb, ch)], dstb)
        pltpu.async_copy(m_hbm.at[srcb.at[0]], rows0, gsem0)
        pltpu.async_copy(m_hbm.at[srcb.at[1]], rows1, gsem1)
        for j in range(ch):
            buf, sem = (rows0, gsem0) if j % 2 == 0 else (rows1, gsem1)
            pltpu.make_async_copy(m_hbm.at[srcb.at[j]], buf, sem).wait()
            pltpu.sync_copy(buf, acc_sh.at[dstb.at[j]], add=True)
            if with_deg:
                pltpu.sync_copy(ones_v, deg_sh.at[dstb.at[j]], add=True)
            if j + 2 < ch:
                pltpu.async_copy(m_hbm.at[srcb.at[j + 2]], buf, sem)
        return carry

    lax.fori_loop(0, nrows // srcb.shape[0], chunk_body, 0)
    plsc.subcore_barrier()

    pltpu.sync_copy(acc_sh.at[pl.ds(s * ZCH, ZCH)],
                    agg_out.at[c, pl.ds(s * ZCH, ZCH)])
    if with_deg:
        pltpu.sync_copy(deg_sh.at[pl.ds(s * ZCH, ZCH)],
                        deg_out.at[c, pl.ds(s * ZCH, ZCH)])


def _make_sc_agg(nrows_per_tile, with_deg):
    mesh = plsc.VectorSubcoreMesh(core_axis_name="c", subcore_axis_name="s",
                                  num_cores=NC, num_subcores=NS)
    del nrows_per_tile
    out_type = [jax.ShapeDtypeStruct((NC, NP, D), jnp.float32)]
    scratch_types = [
        pltpu.VMEM_SHARED((NP, D), jnp.float32),
        pltpu.VMEM((ICH, ROW), jnp.int32),
        pltpu.VMEM((ICH, ROW), jnp.int32),
        pltpu.VMEM((ROW, D), jnp.float32),
        pltpu.VMEM((ROW, D), jnp.float32),
        pltpu.SemaphoreType.DMA,
        pltpu.SemaphoreType.DMA,
    ]
    if with_deg:
        out_type.append(jax.ShapeDtypeStruct((NC, NP), jnp.float32))
        scratch_types += [
            pltpu.VMEM_SHARED((NP,), jnp.float32),
            pltpu.VMEM((ROW,), jnp.float32),
        ]

        def ordered(m, src, dst, zacc, zdeg, agg_out, deg_out,
                    acc_sh, srcb, dstb, rows0, rows1, gsem0, gsem1,
                    deg_sh, ones_v):
            _sc_agg_body(True, m, src, dst, zacc, zdeg, agg_out, deg_out,
                         acc_sh, deg_sh, srcb, dstb, rows0, rows1, ones_v,
                         gsem0, gsem1)
    else:
        def ordered(m, src, dst, zacc, agg_out,
                    acc_sh, srcb, dstb, rows0, rows1, gsem0, gsem1):
            _sc_agg_body(False, m, src, dst, zacc, agg_out,
                         acc_sh, srcb, dstb, rows0, rows1, gsem0, gsem1)

    return pl.kernel(ordered, out_type=out_type, mesh=mesh,
                     scratch_types=scratch_types)


def _tc_in(x_ref, g_ref, b_ref, wi_ref, bi_ref, w1_ref, b1_ref,
           v1_ref, m1_ref):
    x = x_ref[...]
    mu = jnp.mean(x, axis=1, keepdims=True)
    xc = x - mu
    var = jnp.mean(xc * xc, axis=1, keepdims=True)
    xn = xc * lax.rsqrt(var + 1e-5) * g_ref[...] + b_ref[...]
    v1 = jnp.maximum(
        jnp.dot(xn, wi_ref[...], preferred_element_type=jnp.float32)
        + bi_ref[...], 0.0)
    v1_ref[...] = v1
    m1_ref[...] = (jnp.dot(v1, w1_ref[...], preferred_element_type=jnp.float32)
                   + b1_ref[...])


def _tc_mid(v1_ref, m1_ref, agg_ref, deg_ref, w2_ref, b2_ref,
            v2_ref, m2_ref):
    m1 = m1_ref[...]
    a = agg_ref[0] + agg_ref[1] + m1
    deg = deg_ref[0] + deg_ref[1] + 1.0
    out = jnp.maximum(a / deg, 0.0)
    v2 = BETA * v1_ref[...] + (1.0 - BETA) * out
    v2_ref[...] = v2
    m2_ref[...] = (jnp.dot(v2, w2_ref[...], preferred_element_type=jnp.float32)
                   + b2_ref[...])


def _tc_out(v2_ref, m2_ref, agg_ref, deg_ref, t_ref, s_ref):
    m2 = m2_ref[...]
    a = agg_ref[0] + agg_ref[1] + m2
    deg = deg_ref[0] + deg_ref[1] + 1.0
    out = jnp.maximum(a / deg, 0.0)
    t2 = BETA * v2_ref[...] + (1.0 - BETA) * out
    nsq = jnp.sum(t2 * t2, axis=1, keepdims=True)
    n = jnp.maximum(jnp.sqrt(nsq), 1e-7)
    en = jnp.exp(n)
    einv = 1.0 / en
    t_ref[...] = 0.5 * (en + einv)
    s_ref[...] = (0.5 * (en - einv) / n) * t2


def _row_spec(b, d):
    return pl.BlockSpec((b, d), lambda i: (i, 0))


def _full_spec(shape):
    nd = len(shape)
    return pl.BlockSpec(shape, lambda i: (0,) * nd)


def _agg_spec():
    return pl.BlockSpec((NC, BLK, D), lambda i: (0, i, 0))


def _deg_spec():
    return pl.BlockSpec((NC, BLK, 1), lambda i: (0, i, 0))


def kernel(x, edge_index, ln_g, ln_b, W_in, b_in, W1, b1, W2, b2):
    src = edge_index[0].astype(jnp.int32)
    dst = edge_index[1].astype(jnp.int32)
    e = src.shape[0]
    align = NW * ROW * 8
    ep = ((e + align - 1) // align) * align
    pad = ep - e
    src2d = jnp.concatenate(
        [src, jnp.zeros((pad,), jnp.int32)]).reshape(ep // ROW, ROW)
    dst2d = jnp.concatenate(
        [dst, jnp.full((pad,), N, jnp.int32)]).reshape(ep // ROW, ROW)
    nrows_per_tile = (ep // ROW) // NW
    zacc = jnp.zeros((NP, D), jnp.float32)
    zdeg = jnp.zeros((NP,), jnp.float32)

    g2 = ln_g.reshape(1, D)
    bn2 = ln_b.reshape(1, D)
    bi2 = b_in.reshape(1, D)
    b12 = b1.reshape(1, D)
    b22 = b2.reshape(1, D)

    v1, m1 = pl.pallas_call(
        _tc_in,
        grid=(GRID,),
        in_specs=[_row_spec(BLK, D), _full_spec((1, D)), _full_spec((1, D)),
                  _full_spec((D, D)), _full_spec((1, D)),
                  _full_spec((D, D)), _full_spec((1, D))],
        out_specs=[_row_spec(BLK, D), _row_spec(BLK, D)],
        out_shape=[jax.ShapeDtypeStruct((N, D), jnp.float32),
                   jax.ShapeDtypeStruct((N, D), jnp.float32)],
    )(x, g2, bn2, W_in, bi2, W1, b12)

    sc_agg_deg = _make_sc_agg(nrows_per_tile, with_deg=True)
    agg1, degp = sc_agg_deg(m1, src2d, dst2d, zacc, zdeg)
    degp = degp.reshape(NC, NP, 1)

    v2, m2 = pl.pallas_call(
        _tc_mid,
        grid=(GRID,),
        in_specs=[_row_spec(BLK, D), _row_spec(BLK, D),
                  _agg_spec(), _deg_spec(),
                  _full_spec((D, D)), _full_spec((1, D))],
        out_specs=[_row_spec(BLK, D), _row_spec(BLK, D)],
        out_shape=[jax.ShapeDtypeStruct((N, D), jnp.float32),
                   jax.ShapeDtypeStruct((N, D), jnp.float32)],
    )(v1, m1, agg1, degp, W2, b22)

    sc_agg = _make_sc_agg(nrows_per_tile, with_deg=False)
    (agg2,) = sc_agg(m2, src2d, dst2d, zacc)

    t, sp = pl.pallas_call(
        _tc_out,
        grid=(GRID,),
        in_specs=[_row_spec(BLK, D), _row_spec(BLK, D),
                  _agg_spec(), _deg_spec()],
        out_specs=[_row_spec(BLK, 1), _row_spec(BLK, D)],
        out_shape=[jax.ShapeDtypeStruct((N, 1), jnp.float32),
                   jax.ShapeDtypeStruct((N, D), jnp.float32)],
    )(v2, m2, agg2, degp)

    return jnp.concatenate([t, sp], axis=-1)
